# Initial kernel scaffold; baseline (speedup 1.0000x reference)
import jax
import jax.numpy as jnp
from jax.experimental import pallas as pl


def kernel(x, pos, edge_index, batch, params):
    raise NotImplementedError("write your pallas kernel here")


# speedup vs baseline: 1.4366x; 1.4366x over previous
import functools

import jax
import jax.numpy as jnp
from jax import lax
from jax.experimental import pallas as pl
from jax.experimental.pallas import tpu as pltpu
from jax.experimental.pallas import tpu_sc as plsc

N = 50000
E = 800000
G = 512
H = 128

NP = 51200
EP = 802816
TRASH = 50000

NC, NS = 2, 16
NW = NC * NS
BLK = 128
EW = EP // NW
NBLK_W = EW // BLK
ES = EP // NS
NBLK_S = ES // BLK
NPS = NP // NS
NBLK_N = NPS // BLK

BN = 2048
NB_GRID = NP // BN
BE = 2048
EB_GRID = EP // BE

_f32 = jnp.float32
_i32 = jnp.int32


def _silu(v):
    return v * jax.nn.sigmoid(v)


def _embed_body(x_ref, w_ref, b_ref, wa_ref, wb_ref, h_ref, ha_ref, hb_ref):
    h = jnp.dot(x_ref[...], w_ref[...], precision=lax.Precision.HIGHEST, preferred_element_type=_f32) + b_ref[...]
    h_ref[...] = h
    ha_ref[...] = jnp.dot(h, wa_ref[...], precision=lax.Precision.HIGHEST, preferred_element_type=_f32)
    hb_ref[...] = jnp.dot(h, wb_ref[...], precision=lax.Precision.HIGHEST, preferred_element_type=_f32)


def _embed_call(xp, emb_wp, emb_b, wa, wb):
    return pl.pallas_call(
        _embed_body,
        grid=(NB_GRID,),
        in_specs=[
            pl.BlockSpec((BN, 16), lambda i: (i, 0)),
            pl.BlockSpec((16, H), lambda i: (0, 0)),
            pl.BlockSpec((1, H), lambda i: (0, 0)),
            pl.BlockSpec((H, H), lambda i: (0, 0)),
            pl.BlockSpec((H, H), lambda i: (0, 0)),
        ],
        out_specs=[
            pl.BlockSpec((BN, H), lambda i: (i, 0)),
            pl.BlockSpec((BN, H), lambda i: (i, 0)),
            pl.BlockSpec((BN, H), lambda i: (i, 0)),
        ],
        out_shape=[jax.ShapeDtypeStruct((NP, H), _f32)] * 3,
    )(xp, emb_wp, emb_b, wa, wb)


def _node_body(h_ref, agg_ref, nw1_ref, nw2_ref, nb_ref, wa_ref, wb_ref,
               pos_ref, pacc_ref, hn_ref, ha_ref, hb_ref, posn_ref):
    acc = jnp.dot(h_ref[...], nw1_ref[...], precision=lax.Precision.HIGHEST, preferred_element_type=_f32)
    for q in range(4):
        acc = acc + jnp.dot(agg_ref[q], nw2_ref[q], precision=lax.Precision.HIGHEST, preferred_element_type=_f32)
    hn = _silu(acc + nb_ref[...])
    hn_ref[...] = hn
    ha_ref[...] = jnp.dot(hn, wa_ref[...], precision=lax.Precision.HIGHEST, preferred_element_type=_f32)
    hb_ref[...] = jnp.dot(hn, wb_ref[...], precision=lax.Precision.HIGHEST, preferred_element_type=_f32)
    posn_ref[...] = pos_ref[...] + pacc_ref[0] + pacc_ref[1]


def _node_call(h, agg4, nw1, nw2q, nb, wa, wb, pos, pacc):
    return pl.pallas_call(
        _node_body,
        grid=(NB_GRID,),
        in_specs=[
            pl.BlockSpec((BN, H), lambda i: (i, 0)),
            pl.BlockSpec((4, BN, 32), lambda i: (0, i, 0)),
            pl.BlockSpec((H, H), lambda i: (0, 0)),
            pl.BlockSpec((4, 32, H), lambda i: (0, 0, 0)),
            pl.BlockSpec((1, H), lambda i: (0, 0)),
            pl.BlockSpec((H, H), lambda i: (0, 0)),
            pl.BlockSpec((H, H), lambda i: (0, 0)),
            pl.BlockSpec((BN, 16), lambda i: (i, 0)),
            pl.BlockSpec((2, BN, 16), lambda i: (0, i, 0)),
        ],
        out_specs=[
            pl.BlockSpec((BN, H), lambda i: (i, 0)),
            pl.BlockSpec((BN, H), lambda i: (i, 0)),
            pl.BlockSpec((BN, H), lambda i: (i, 0)),
            pl.BlockSpec((BN, 16), lambda i: (i, 0)),
        ],
        out_shape=[
            jax.ShapeDtypeStruct((NP, H), _f32),
            jax.ShapeDtypeStruct((NP, H), _f32),
            jax.ShapeDtypeStruct((NP, H), _f32),
            jax.ShapeDtypeStruct((NP, 16), _f32),
        ],
    )(h, agg4, nw1, nw2q, nb, wa, wb, pos, pacc)


def _edge_body(pre_ref, posr_ref, posc_ref, w1c_ref, b1_ref, w2_ref, b2_ref,
               cw_ref, cb_ref, m4_ref, cu_ref):
    diff = posr_ref[...] - posc_ref[...]
    dist2 = jnp.sum(diff * diff, axis=1, keepdims=True)
    pre = pre_ref[...] + dist2 * w1c_ref[...] + b1_ref[...]
    m1 = _silu(pre)
    m = _silu(jnp.dot(m1, w2_ref[...], precision=lax.Precision.HIGHEST, preferred_element_type=_f32) + b2_ref[...])
    s = jnp.sum(m * cw_ref[...], axis=1, keepdims=True) + cb_ref[0, 0]
    cu_ref[...] = diff * jnp.tanh(s)
    for q in range(4):
        m4_ref[q] = m[:, q * 32:(q + 1) * 32]


def _edge_call(pre, posr, posc, w1c, b1, w2, b2, cw, cb):
    return pl.pallas_call(
        _edge_body,
        grid=(EB_GRID,),
        in_specs=[
            pl.BlockSpec((BE, H), lambda i: (i, 0)),
            pl.BlockSpec((BE, 16), lambda i: (i, 0)),
            pl.BlockSpec((BE, 16), lambda i: (i, 0)),
            pl.BlockSpec((1, H), lambda i: (0, 0)),
            pl.BlockSpec((1, H), lambda i: (0, 0)),
            pl.BlockSpec((H, H), lambda i: (0, 0)),
            pl.BlockSpec((1, H), lambda i: (0, 0)),
            pl.BlockSpec((1, H), lambda i: (0, 0)),
            pl.BlockSpec(memory_space=pltpu.SMEM),
        ],
        out_specs=[
            pl.BlockSpec((4, BE, 32), lambda i: (0, i, 0)),
            pl.BlockSpec((BE, 16), lambda i: (i, 0)),
        ],
        out_shape=[
            jax.ShapeDtypeStruct((4, EP, 32), _f32),
            jax.ShapeDtypeStruct((EP, 16), _f32),
        ],
    )(pre, posr, posc, w1c, b1, w2, b2, cw, cb)


def _pool_body(b_ref, h_ref, seg_ref, cnt_ref):
    i = pl.program_id(0)

    @pl.when(i == 0)
    def _():
        seg_ref[...] = jnp.zeros((G, H), _f32)
        cnt_ref[...] = jnp.zeros((G, H), _f32)

    gids = lax.broadcasted_iota(_i32, (G, BN), 0)
    onehot = (gids == jnp.broadcast_to(b_ref[0], (G, BN))).astype(_f32)
    seg_ref[...] += jnp.dot(onehot, h_ref[...], precision=lax.Precision.HIGHEST, preferred_element_type=_f32)
    cnt = jnp.sum(onehot, axis=1, keepdims=True)
    cnt_ref[...] += jnp.broadcast_to(cnt, (G, H))


def _pool_call(batch3, h):
    return pl.pallas_call(
        _pool_body,
        grid=(NB_GRID,),
        in_specs=[
            pl.BlockSpec((1, 1, BN), lambda i: (i, 0, 0)),
            pl.BlockSpec((BN, H), lambda i: (i, 0)),
        ],
        out_specs=[
            pl.BlockSpec((G, H), lambda i: (0, 0)),
            pl.BlockSpec((G, H), lambda i: (0, 0)),
        ],
        out_shape=[jax.ShapeDtypeStruct((G, H), _f32)] * 2,
    )(batch3, h)


def _head_body(seg_ref, cnt_ref, w1_ref, b1_ref, w2_ref, b2_ref, out_ref):
    mean = seg_ref[...] / jnp.maximum(cnt_ref[...], 1.0)
    o = _silu(jnp.dot(mean, w1_ref[...], precision=lax.Precision.HIGHEST, preferred_element_type=_f32) + b1_ref[...])
    out_ref[...] = jnp.dot(o, w2_ref[...], precision=lax.Precision.HIGHEST, preferred_element_type=_f32) + b2_ref[...]


def _head_call(seg, cnt, h1w, h1b, h2wp, h2b):
    return pl.pallas_call(
        _head_body,
        out_shape=jax.ShapeDtypeStruct((G, H), _f32),
    )(seg, cnt, h1w, h1b, h2wp, h2b)


@functools.cache
def _mesh():
    return plsc.VectorSubcoreMesh(core_axis_name="c", subcore_axis_name="s",
                                  num_cores=NC, num_subcores=NS)


def _gather_body(hA, hB, pos4, row, col, pre_out, posr_out, posc_out,
                 idxr, idxc, bufA, bufB, bufPr, bufPc, semA, semB, semP):
    wid = lax.axis_index("s") * NC + lax.axis_index("c")
    base = wid * EW

    def step(nb, _):
        off = base + nb * BLK
        pltpu.sync_copy(row.at[pl.ds(off, BLK)], idxr)
        pltpu.sync_copy(col.at[pl.ds(off, BLK)], idxc)
        cpA = pltpu.async_copy(hA.at[idxr], bufA, semA)
        cpB = pltpu.async_copy(hB.at[idxc], bufB, semB)
        cpPr = pltpu.async_copy(pos4.at[idxr], bufPr, semP)
        cpPc = pltpu.async_copy(pos4.at[idxc], bufPc, semP)
        cpA.wait()
        cpB.wait()
        cpPr.wait()
        cpPc.wait()

        def rowstep(i, _):
            for j in range(8):
                sl = pl.ds(j * 16, 16)
                bufA[i, sl] = bufA[i, sl] + bufB[i, sl]
            return 0

        lax.fori_loop(0, BLK, rowstep, 0)
        pltpu.sync_copy(bufA, pre_out.at[pl.ds(off, BLK)])
        pltpu.sync_copy(bufPr, posr_out.at[pl.ds(off, BLK)])
        pltpu.sync_copy(bufPc, posc_out.at[pl.ds(off, BLK)])
        return 0

    lax.fori_loop(0, NBLK_W, step, 0)


@functools.cache
def _gather_kernel():
    return pl.kernel(
        _gather_body,
        out_type=(
            jax.ShapeDtypeStruct((EP, H), _f32),
            jax.ShapeDtypeStruct((EP, 16), _f32),
            jax.ShapeDtypeStruct((EP, 16), _f32),
        ),
        mesh=_mesh(),
        compiler_params=pltpu.CompilerParams(use_tc_tiling_on_sc=False),
        scratch_types=[
        pltpu.VMEM((BLK,), _i32),
        pltpu.VMEM((BLK,), _i32),
        pltpu.VMEM((BLK, H), _f32),
        pltpu.VMEM((BLK, H), _f32),
        pltpu.VMEM((BLK, 16), _f32),
        pltpu.VMEM((BLK, 16), _f32),
            pltpu.SemaphoreType.DMA,
            pltpu.SemaphoreType.DMA,
            pltpu.SemaphoreType.DMA,
        ],
    )


def _scatter_body(m4f, row, z32, agg_out, spA, idx, mb):
    cid = lax.axis_index("c")
    sid = lax.axis_index("s")

    for p in range(2):
        pltpu.sync_copy(z32, spA.at[pl.ds(sid * NPS, NPS)])

        plsc.subcore_barrier()

        q = cid * 2 + p

        def estep(nb, _):
            off = sid * ES + nb * BLK
            pltpu.sync_copy(row.at[pl.ds(off, BLK)], idx)
            pltpu.sync_copy(m4f.at[pl.ds(q * EP + off, BLK)], mb)
            pltpu.sync_copy(mb, spA.at[idx], add=True)
            return 0

        lax.fori_loop(0, NBLK_S, estep, 0)
        plsc.subcore_barrier()

        def cstep(i, _):
            r0 = sid * NPS + i * BLK
            pltpu.sync_copy(spA.at[pl.ds(r0, BLK)],
                            agg_out.at[pl.ds(q * NP + r0, BLK)])
            return 0

        lax.fori_loop(0, NBLK_N, cstep, 0)
        plsc.subcore_barrier()


@functools.cache
def _scatter_kernel():
    return pl.kernel(
        _scatter_body,
        out_type=jax.ShapeDtypeStruct((4 * NP, 32), _f32),
        mesh=_mesh(),
        compiler_params=pltpu.CompilerParams(use_tc_tiling_on_sc=False),
        scratch_types=[
            pltpu.VMEM_SHARED((NP, 32), _f32),
            pltpu.VMEM((BLK,), _i32),
            pltpu.VMEM((BLK, 32), _f32),
        ],
    )


CW = 16


def _cuscatter_body(cu, row, zc, pacc_out, spP, idx, cub):
    cid = lax.axis_index("c")
    sid = lax.axis_index("s")

    pltpu.sync_copy(zc, spP.at[pl.ds(sid * NPS, NPS)])
    plsc.subcore_barrier()

    base = (cid * NS + sid) * EW

    def estep(nb, _):
        off = base + nb * BLK
        pltpu.sync_copy(row.at[pl.ds(off, BLK)], idx)
        pltpu.sync_copy(cu.at[pl.ds(off, BLK)], cub)
        pltpu.sync_copy(cub, spP.at[idx], add=True)
        return 0

    lax.fori_loop(0, NBLK_W, estep, 0)
    plsc.subcore_barrier()

    def cstep(i, _):
        r0 = sid * NPS + i * BLK
        pltpu.sync_copy(spP.at[pl.ds(r0, BLK)],
                        pacc_out.at[pl.ds(cid * NP + r0, BLK)])
        return 0

    lax.fori_loop(0, NBLK_N, cstep, 0)


@functools.cache
def _cuscatter_kernel():
    return pl.kernel(
        _cuscatter_body,
        out_type=jax.ShapeDtypeStruct((2 * NP, CW), _f32),
        mesh=_mesh(),
        compiler_params=pltpu.CompilerParams(use_tc_tiling_on_sc=False),
        scratch_types=[
            pltpu.VMEM_SHARED((NP, CW), _f32),
            pltpu.VMEM((BLK,), _i32),
            pltpu.VMEM((BLK, CW), _f32),
        ],
    )


def kernel(x, pos, edge_index, batch, params):
    row = jnp.pad(edge_index[0].astype(_i32), (0, EP - E), constant_values=TRASH)
    col = jnp.pad(edge_index[1].astype(_i32), (0, EP - E), constant_values=TRASH)
    pos16 = jnp.pad(pos.astype(_f32), ((0, NP - N), (0, 13)))
    xp = jnp.pad(x.astype(_f32), ((0, NP - N), (0, 16 - x.shape[1])))
    batch3 = jnp.pad(batch.astype(_i32), (0, NP - N),
                     constant_values=G).reshape(NB_GRID, 1, BN)

    p = params
    emb_wp = jnp.pad(p['emb_w'], ((0, 16 - p['emb_w'].shape[0]), (0, 0)))
    emb_b = p['emb_b'].reshape(1, H)

    lyr = p['layers']
    w1a = [l['e_w1'][:H] for l in lyr]
    w1b = [l['e_w1'][H:2 * H] for l in lyr]
    w1c = [l['e_w1'][2 * H:].reshape(1, H) for l in lyr]
    b1 = [l['e_b1'].reshape(1, H) for l in lyr]
    w2 = [l['e_w2'] for l in lyr]
    b2 = [l['e_b2'].reshape(1, H) for l in lyr]
    cw = [l['c_w'].reshape(1, H) for l in lyr]
    cb = [l['c_b'].reshape(1, 1) for l in lyr]
    nw1 = [l['n_w'][:H] for l in lyr]
    nw2q = [l['n_w'][H:].reshape(4, 32, H) for l in lyr]
    nb = [l['n_b'].reshape(1, H) for l in lyr]
    zW = jnp.zeros((H, H), _f32)

    h, hA, hB = _embed_call(xp, emb_wp, emb_b, w1a[0], w1b[0])
    pos_cur = pos16
    for i in range(4):
        pre, posr, posc = _gather_kernel()(hA, hB, pos_cur, row, col)
        m4, cu = _edge_call(pre, posr, posc, w1c[i], b1[i], w2[i], b2[i],
                            cw[i], cb[i])
        m4f = m4.reshape(4 * EP, 32)
        z32 = jnp.zeros((NPS, 32), _f32)
        zc = jnp.zeros((NPS, CW), _f32)
        aggf = _scatter_kernel()(m4f, row, z32)
        paccf = _cuscatter_kernel()(cu, row, zc)
        agg4 = aggf.reshape(4, NP, 32)
        pacc2 = paccf.reshape(2, NP, CW)
        wa, wb = (w1a[i + 1], w1b[i + 1]) if i < 3 else (zW, zW)
        h, hA, hB, pos_cur = _node_call(h, agg4, nw1[i], nw2q[i], nb[i],
                                        wa, wb, pos_cur, pacc2)

    seg, cnt = _pool_call(batch3, h)
    h2wp = jnp.pad(p['h2_w'], ((0, 0), (0, H - 1)))
    h2b = jnp.broadcast_to(p['h2_b'].reshape(1, 1), (1, H))
    res = _head_call(seg, cnt, p['h1_w'], p['h1_b'].reshape(1, H), h2wp, h2b)
    return res[:, 0]
